# baseline (speedup 1.0000x reference)
import functools

import jax
import jax.numpy as jnp
from jax import lax
from jax.experimental import pallas as pl
from jax.experimental.pallas import tpu as pltpu
from jax.experimental.pallas import tpu_sc as plsc

_KTOP = 5
_LANES = 128
_KB = 1024


def _ce(a, b):
    return jnp.maximum(a, b), jnp.minimum(a, b)


def _insert(r, chunks):
    assert len(chunks) % 4 == 0
    for t in range(len(chunks) // 4):
        y = list(chunks[4 * t:4 * t + 4])
        y[0], y[1] = _ce(y[0], y[1])
        y[2], y[3] = _ce(y[2], y[3])
        y[0], y[2] = _ce(y[0], y[2])
        y[1], y[3] = _ce(y[1], y[3])
        y[1], y[2] = _ce(y[1], y[2])
        c = [r[0],
             jnp.maximum(r[1], y[3]),
             jnp.maximum(r[2], y[2]),
             jnp.maximum(r[3], y[1]),
             jnp.maximum(r[4], y[0])]
        c[0], c[4] = _ce(c[0], c[4])
        c[1], c[3] = _ce(c[1], c[3])
        c[2], c[4] = _ce(c[2], c[4])
        c[1], c[2] = _ce(c[1], c[2])
        c[3], c[4] = _ce(c[3], c[4])
        r = c
    return r


def _m_chunks(m):
    return [m[:, c * _LANES:(c + 1) * _LANES]
            for c in range(m.shape[1] // _LANES)]


def _tc_body(q_ref, k_ref, cand_ref, qsq_ref, r_ref, mprev_ref, *, nt, ktop):
    j = pl.program_id(0)

    @pl.when(j == 0)
    def _init():
        r_ref[...] = jnp.full(r_ref.shape, -jnp.inf, dtype=r_ref.dtype)
        mprev_ref[...] = jnp.full(mprev_ref.shape, -jnp.inf,
                                  dtype=mprev_ref.dtype)

    @pl.when(j < nt)
    def _main():
        q = q_ref[...]

        def mk(kblk):
            kb16 = kblk.astype(jnp.bfloat16)
            cross = jax.lax.dot_general(
                q, kb16, (((1,), (1,)), ((), ())),
                preferred_element_type=jnp.float32,
            )
            ksq = jnp.sum(kblk * kblk, axis=1)
            return cross - 0.5 * ksq[None, :]

        r = [r_ref[i] for i in range(ktop)]
        r = _insert(r, _m_chunks(mprev_ref[...]))
        m_a = mk(k_ref[:_KB, :])
        r = _insert(r, _m_chunks(m_a))
        m_b = mk(k_ref[_KB:, :])
        for i in range(ktop):
            r_ref[i] = r[i]
        mprev_ref[...] = m_b

    @pl.when(j == nt)
    def _final():
        r = [r_ref[i] for i in range(ktop)]
        r = _insert(r, _m_chunks(mprev_ref[...]))
        for i in range(ktop):
            cand_ref[i] = r[i]
        qf = q_ref[...].astype(jnp.float32)
        qsq_ref[...] = jnp.sum(qf * qf, axis=1, keepdims=True)


def _sc_sqrt(x):
    i = lax.bitcast_convert_type(x, jnp.int32)
    i = jnp.int32(0x5F3759DF) - lax.shift_right_arithmetic(i, 1)
    y = lax.bitcast_convert_type(i, jnp.float32)
    for _ in range(3):
        y = y * (1.5 - 0.5 * x * y * y)
    return x * y


def _sc_merge_body(arr_hbm, qsq_hbm, out_hbm, grp_v, qsq_v, out_v, *,
                   ktop, rows_per_w):
    nc = 2
    wid = lax.axis_index("s") * nc + lax.axis_index("c")
    base = wid * rows_per_w
    ngroups = rows_per_w // 16
    for gg in range(ngroups):
        pltpu.sync_copy(arr_hbm.at[wid * ngroups + gg], grp_v.at[gg])
    pltpu.sync_copy(qsq_hbm.at[pl.ds(base, rows_per_w)], qsq_v)

    for gg in range(ngroups):
        ninf16 = jnp.full((16,), -jnp.inf, jnp.float32)

        def quad_body(qt, r, gg=gg):
            y = [grp_v[gg, 4 * qt + qi, :] for qi in range(4)]
            r = list(r)
            y[0], y[1] = _ce(y[0], y[1])
            y[2], y[3] = _ce(y[2], y[3])
            y[0], y[2] = _ce(y[0], y[2])
            y[1], y[3] = _ce(y[1], y[3])
            y[1], y[2] = _ce(y[1], y[2])
            c = [r[0],
                 jnp.maximum(r[1], y[3]),
                 jnp.maximum(r[2], y[2]),
                 jnp.maximum(r[3], y[1]),
                 jnp.maximum(r[4], y[0])]
            c[0], c[4] = _ce(c[0], c[4])
            c[1], c[3] = _ce(c[1], c[3])
            c[2], c[4] = _ce(c[2], c[4])
            c[1], c[2] = _ce(c[1], c[2])
            c[3], c[4] = _ce(c[3], c[4])
            return tuple(c)

        r = lax.fori_loop(0, (ktop * _LANES) // 4, quad_body,
                          (ninf16,) * ktop)
        qv = qsq_v[gg * 16:(gg + 1) * 16]
        score = jnp.zeros((16,), jnp.float32)
        for i in range(ktop):
            d2 = jnp.maximum(qv - 2.0 * r[i], 0.0)
            score = score + _sc_sqrt(d2 + 1e-12)
        out_v[gg * 16:(gg + 1) * 16] = score
    pltpu.sync_copy(out_v, out_hbm.at[pl.ds(base, rows_per_w)])


def kernel(queries, keys, k):
    q_rows, d = queries.shape
    n_keys = keys.shape[0]
    nt = n_keys // (2 * _KB)

    qb16 = queries.astype(jnp.bfloat16)

    cand, qsq = pl.pallas_call(
        functools.partial(_tc_body, nt=nt, ktop=_KTOP),
        grid=(nt + 1,),
        in_specs=[
            pl.BlockSpec((q_rows, d), lambda j: (0, 0)),
            pl.BlockSpec((2 * _KB, d), lambda j: (jnp.minimum(j, nt - 1), 0)),
        ],
        out_specs=[
            pl.BlockSpec((_KTOP, q_rows, _LANES), lambda j: (0, 0, 0)),
            pl.BlockSpec((q_rows, 1), lambda j: (0, 0)),
        ],
        out_shape=[
            jax.ShapeDtypeStruct((_KTOP, q_rows, _LANES), jnp.float32),
            jax.ShapeDtypeStruct((q_rows, 1), jnp.float32),
        ],
        scratch_shapes=[
            pltpu.VMEM((_KTOP, q_rows, _LANES), jnp.float32),
            pltpu.VMEM((q_rows, _KB), jnp.float32),
        ],
    )(qb16, keys)

    n_workers = 32
    rows_per_w = q_rows // n_workers
    ngroups = rows_per_w // 16
    width = _KTOP * _LANES
    arr = (cand.transpose(1, 0, 2).reshape(q_rows // 16, 16, width)
           .transpose(0, 2, 1))
    mesh = plsc.VectorSubcoreMesh(core_axis_name="c", subcore_axis_name="s")
    scores = pl.kernel(
        functools.partial(_sc_merge_body, ktop=_KTOP,
                          rows_per_w=rows_per_w),
        mesh=mesh,
        compiler_params=pltpu.CompilerParams(use_tc_tiling_on_sc=False),
        out_type=jax.ShapeDtypeStruct((q_rows,), jnp.float32),
        scratch_types=[
            pltpu.VMEM((ngroups, width, 16), jnp.float32),
            pltpu.VMEM((rows_per_w,), jnp.float32),
            pltpu.VMEM((rows_per_w,), jnp.float32),
        ],
    )(arr, qsq.reshape(q_rows))
    return scores / k


# speedup vs baseline: 1.0377x; 1.0377x over previous
import functools

import jax
import jax.numpy as jnp
from jax import lax
from jax.experimental import pallas as pl
from jax.experimental.pallas import tpu as pltpu
from jax.experimental.pallas import tpu_sc as plsc

_KTOP = 5
_LANES = 128
_KB = 1024


def _ce(a, b):
    return jnp.maximum(a, b), jnp.minimum(a, b)


def _insert(r, chunks):
    assert len(chunks) % 4 == 0
    for t in range(len(chunks) // 4):
        y = list(chunks[4 * t:4 * t + 4])
        y[0], y[1] = _ce(y[0], y[1])
        y[2], y[3] = _ce(y[2], y[3])
        y[0], y[2] = _ce(y[0], y[2])
        y[1], y[3] = _ce(y[1], y[3])
        y[1], y[2] = _ce(y[1], y[2])
        c = [r[0],
             jnp.maximum(r[1], y[3]),
             jnp.maximum(r[2], y[2]),
             jnp.maximum(r[3], y[1]),
             jnp.maximum(r[4], y[0])]
        c[0], c[4] = _ce(c[0], c[4])
        c[1], c[3] = _ce(c[1], c[3])
        c[2], c[4] = _ce(c[2], c[4])
        c[1], c[2] = _ce(c[1], c[2])
        c[3], c[4] = _ce(c[3], c[4])
        r = c
    return r


def _m_chunks(m):
    return [m[:, c * _LANES:(c + 1) * _LANES]
            for c in range(m.shape[1] // _LANES)]


def _tc_body(q_ref, k_ref, cand_ref, qsq_ref, r_ref, mprev_ref, *, nt, ktop):
    j = pl.program_id(0)

    @pl.when(j == 0)
    def _init():
        r_ref[...] = jnp.full(r_ref.shape, -jnp.inf, dtype=r_ref.dtype)
        mprev_ref[...] = jnp.full(mprev_ref.shape, -jnp.inf,
                                  dtype=mprev_ref.dtype)

    @pl.when(j < nt)
    def _main():
        q = q_ref[...]

        def mk(kblk):
            kb16 = kblk.astype(jnp.bfloat16)
            cross = jax.lax.dot_general(
                q, kb16, (((1,), (1,)), ((), ())),
                preferred_element_type=jnp.float32,
            )
            ksq = jnp.sum(kblk * kblk, axis=1)
            return cross - 0.5 * ksq[None, :]

        r = [r_ref[i] for i in range(ktop)]
        r = _insert(r, _m_chunks(mprev_ref[...]))
        m_a = mk(k_ref[:_KB, :])
        r = _insert(r, _m_chunks(m_a))
        m_b = mk(k_ref[_KB:, :])
        for i in range(ktop):
            r_ref[i] = r[i]
        mprev_ref[...] = m_b

    @pl.when(j == nt)
    def _final():
        r = [r_ref[i] for i in range(ktop)]
        r = _insert(r, _m_chunks(mprev_ref[...]))
        q_rows = r[0].shape[0]
        for i in range(ktop):
            v = r[i].reshape(q_rows // 16, 16, _LANES)
            cand_ref[:, i * _LANES:(i + 1) * _LANES, :] = jnp.swapaxes(v, 1, 2)
        qf = q_ref[...].astype(jnp.float32)
        qsq_ref[...] = jnp.sum(qf * qf, axis=1, keepdims=True)


def _sc_sqrt(x):
    i = lax.bitcast_convert_type(x, jnp.int32)
    i = jnp.int32(0x5F3759DF) - lax.shift_right_arithmetic(i, 1)
    y = lax.bitcast_convert_type(i, jnp.float32)
    for _ in range(3):
        y = y * (1.5 - 0.5 * x * y * y)
    return x * y


def _sc_merge_body(arr_hbm, qsq_hbm, out_hbm, grp_v, qsq_v, out_v, *,
                   ktop, rows_per_w):
    nc = 2
    wid = lax.axis_index("s") * nc + lax.axis_index("c")
    base = wid * rows_per_w
    ngroups = rows_per_w // 16
    for gg in range(ngroups):
        pltpu.sync_copy(arr_hbm.at[wid * ngroups + gg], grp_v.at[gg])
    pltpu.sync_copy(qsq_hbm.at[pl.ds(base, rows_per_w)], qsq_v)

    for gg in range(ngroups):
        ninf16 = jnp.full((16,), -jnp.inf, jnp.float32)

        def quad_body(qt, r, gg=gg):
            y = [grp_v[gg, 4 * qt + qi, :] for qi in range(4)]
            r = list(r)
            y[0], y[1] = _ce(y[0], y[1])
            y[2], y[3] = _ce(y[2], y[3])
            y[0], y[2] = _ce(y[0], y[2])
            y[1], y[3] = _ce(y[1], y[3])
            y[1], y[2] = _ce(y[1], y[2])
            c = [r[0],
                 jnp.maximum(r[1], y[3]),
                 jnp.maximum(r[2], y[2]),
                 jnp.maximum(r[3], y[1]),
                 jnp.maximum(r[4], y[0])]
            c[0], c[4] = _ce(c[0], c[4])
            c[1], c[3] = _ce(c[1], c[3])
            c[2], c[4] = _ce(c[2], c[4])
            c[1], c[2] = _ce(c[1], c[2])
            c[3], c[4] = _ce(c[3], c[4])
            return tuple(c)

        r = lax.fori_loop(0, (ktop * _LANES) // 4, quad_body,
                          (ninf16,) * ktop)
        qv = qsq_v[gg * 16:(gg + 1) * 16]
        score = jnp.zeros((16,), jnp.float32)
        for i in range(ktop):
            d2 = jnp.maximum(qv - 2.0 * r[i], 0.0)
            score = score + _sc_sqrt(d2 + 1e-12)
        out_v[gg * 16:(gg + 1) * 16] = score
    pltpu.sync_copy(out_v, out_hbm.at[pl.ds(base, rows_per_w)])


def kernel(queries, keys, k):
    q_rows, d = queries.shape
    n_keys = keys.shape[0]
    nt = n_keys // (2 * _KB)

    qb16 = queries.astype(jnp.bfloat16)

    cand, qsq = pl.pallas_call(
        functools.partial(_tc_body, nt=nt, ktop=_KTOP),
        grid=(nt + 1,),
        in_specs=[
            pl.BlockSpec((q_rows, d), lambda j: (0, 0)),
            pl.BlockSpec((2 * _KB, d), lambda j: (jnp.minimum(j, nt - 1), 0)),
        ],
        out_specs=[
            pl.BlockSpec((q_rows // 16, _KTOP * _LANES, 16),
                         lambda j: (0, 0, 0)),
            pl.BlockSpec((q_rows, 1), lambda j: (0, 0)),
        ],
        out_shape=[
            jax.ShapeDtypeStruct((q_rows // 16, _KTOP * _LANES, 16),
                                 jnp.float32),
            jax.ShapeDtypeStruct((q_rows, 1), jnp.float32),
        ],
        scratch_shapes=[
            pltpu.VMEM((_KTOP, q_rows, _LANES), jnp.float32),
            pltpu.VMEM((q_rows, _KB), jnp.float32),
        ],
    )(qb16, keys)

    n_workers = 32
    rows_per_w = q_rows // n_workers
    ngroups = rows_per_w // 16
    width = _KTOP * _LANES
    arr = cand
    mesh = plsc.VectorSubcoreMesh(core_axis_name="c", subcore_axis_name="s")
    scores = pl.kernel(
        functools.partial(_sc_merge_body, ktop=_KTOP,
                          rows_per_w=rows_per_w),
        mesh=mesh,
        compiler_params=pltpu.CompilerParams(use_tc_tiling_on_sc=False),
        out_type=jax.ShapeDtypeStruct((q_rows,), jnp.float32),
        scratch_types=[
            pltpu.VMEM((ngroups, width, 16), jnp.float32),
            pltpu.VMEM((rows_per_w,), jnp.float32),
            pltpu.VMEM((rows_per_w,), jnp.float32),
        ],
    )(arr, qsq.reshape(q_rows))
    return scores / k


# speedup vs baseline: 1.0505x; 1.0124x over previous
import functools

import jax
import jax.numpy as jnp
from jax import lax
from jax.experimental import pallas as pl
from jax.experimental.pallas import tpu as pltpu
from jax.experimental.pallas import tpu_sc as plsc

_KTOP = 5
_LANES = 128
_KB = 1024


def _ce(a, b):
    return jnp.maximum(a, b), jnp.minimum(a, b)


def _insert(r, chunks):
    assert len(chunks) % 4 == 0
    for t in range(len(chunks) // 4):
        y = list(chunks[4 * t:4 * t + 4])
        y[0], y[1] = _ce(y[0], y[1])
        y[2], y[3] = _ce(y[2], y[3])
        y[0], y[2] = _ce(y[0], y[2])
        y[1], y[3] = _ce(y[1], y[3])
        y[1], y[2] = _ce(y[1], y[2])
        c = [r[0],
             jnp.maximum(r[1], y[3]),
             jnp.maximum(r[2], y[2]),
             jnp.maximum(r[3], y[1]),
             jnp.maximum(r[4], y[0])]
        c[0], c[4] = _ce(c[0], c[4])
        c[1], c[3] = _ce(c[1], c[3])
        c[2], c[4] = _ce(c[2], c[4])
        c[1], c[2] = _ce(c[1], c[2])
        c[3], c[4] = _ce(c[3], c[4])
        r = c
    return r


def _m_chunks(m):
    return [m[:, c * _LANES:(c + 1) * _LANES]
            for c in range(m.shape[1] // _LANES)]


def _tc_body(q_ref, k_ref, cand_ref, r_ref, mprev_ref, *, nt, ktop):
    j = pl.program_id(0)

    @pl.when(j == 0)
    def _init():
        r_ref[...] = jnp.full(r_ref.shape, -jnp.inf, dtype=r_ref.dtype)
        mprev_ref[...] = jnp.full(mprev_ref.shape, -jnp.inf,
                                  dtype=mprev_ref.dtype)

    @pl.when(j < nt)
    def _main():
        q = q_ref[...]

        def mk(kblk):
            kb16 = kblk.astype(jnp.bfloat16)
            cross = jax.lax.dot_general(
                q, kb16, (((1,), (1,)), ((), ())),
                preferred_element_type=jnp.float32,
            )
            ksq = jnp.sum(kblk * kblk, axis=1)
            return cross - 0.5 * ksq[None, :]

        r = [r_ref[i] for i in range(ktop)]
        r = _insert(r, _m_chunks(mprev_ref[...]))
        m_a = mk(k_ref[:_KB, :])
        r = _insert(r, _m_chunks(m_a))
        m_b = mk(k_ref[_KB:, :])
        for i in range(ktop):
            r_ref[i] = r[i]
        mprev_ref[...] = m_b

    @pl.when(j == nt)
    def _final():
        r = [r_ref[i] for i in range(ktop)]
        r = _insert(r, _m_chunks(mprev_ref[...]))
        q_rows = r[0].shape[0]
        qf = q_ref[...].astype(jnp.float32)
        qsq = jnp.sum(qf * qf, axis=1, keepdims=True)
        for i in range(ktop):
            v = (2.0 * r[i] - qsq).reshape(q_rows // 16, 16, _LANES)
            cand_ref[:, i * _LANES:(i + 1) * _LANES, :] = jnp.swapaxes(v, 1, 2)


def _sc_sqrt(x):
    i = lax.bitcast_convert_type(x, jnp.int32)
    i = jnp.int32(0x5F3759DF) - lax.shift_right_arithmetic(i, 1)
    y = lax.bitcast_convert_type(i, jnp.float32)
    for _ in range(3):
        y = y * (1.5 - 0.5 * x * y * y)
    return x * y


def _sc_merge_body(arr_hbm, out_hbm, grp_v, out_v, *, ktop, rows_per_w):
    nc = 2
    wid = lax.axis_index("s") * nc + lax.axis_index("c")
    base = wid * rows_per_w
    ngroups = rows_per_w // 16
    for gg in range(ngroups):
        pltpu.sync_copy(arr_hbm.at[wid * ngroups + gg], grp_v.at[gg])

    for gg in range(ngroups):
        ninf16 = jnp.full((16,), -jnp.inf, jnp.float32)

        def quad_body(qt, r, gg=gg):
            y = [grp_v[gg, 4 * qt + qi, :] for qi in range(4)]
            r = list(r)
            y[0], y[1] = _ce(y[0], y[1])
            y[2], y[3] = _ce(y[2], y[3])
            y[0], y[2] = _ce(y[0], y[2])
            y[1], y[3] = _ce(y[1], y[3])
            y[1], y[2] = _ce(y[1], y[2])
            c = [r[0],
                 jnp.maximum(r[1], y[3]),
                 jnp.maximum(r[2], y[2]),
                 jnp.maximum(r[3], y[1]),
                 jnp.maximum(r[4], y[0])]
            c[0], c[4] = _ce(c[0], c[4])
            c[1], c[3] = _ce(c[1], c[3])
            c[2], c[4] = _ce(c[2], c[4])
            c[1], c[2] = _ce(c[1], c[2])
            c[3], c[4] = _ce(c[3], c[4])
            return tuple(c)

        r = lax.fori_loop(0, (ktop * _LANES) // 4, quad_body,
                          (ninf16,) * ktop)
        score = jnp.zeros((16,), jnp.float32)
        for i in range(ktop):
            d2 = jnp.maximum(-r[i], 0.0)
            score = score + _sc_sqrt(d2 + 1e-12)
        out_v[gg * 16:(gg + 1) * 16] = score
    pltpu.sync_copy(out_v, out_hbm.at[pl.ds(base, rows_per_w)])


def kernel(queries, keys, k):
    q_rows, d = queries.shape
    n_keys = keys.shape[0]
    nt = n_keys // (2 * _KB)

    qb16 = queries.astype(jnp.bfloat16)

    cand = pl.pallas_call(
        functools.partial(_tc_body, nt=nt, ktop=_KTOP),
        grid=(nt + 1,),
        in_specs=[
            pl.BlockSpec((q_rows, d), lambda j: (0, 0)),
            pl.BlockSpec((2 * _KB, d), lambda j: (jnp.minimum(j, nt - 1), 0)),
        ],
        out_specs=pl.BlockSpec((q_rows // 16, _KTOP * _LANES, 16),
                               lambda j: (0, 0, 0)),
        out_shape=jax.ShapeDtypeStruct((q_rows // 16, _KTOP * _LANES, 16),
                                       jnp.float32),
        scratch_shapes=[
            pltpu.VMEM((_KTOP, q_rows, _LANES), jnp.float32),
            pltpu.VMEM((q_rows, _KB), jnp.float32),
        ],
    )(qb16, keys)

    n_workers = 32
    rows_per_w = q_rows // n_workers
    ngroups = rows_per_w // 16
    width = _KTOP * _LANES
    arr = cand
    mesh = plsc.VectorSubcoreMesh(core_axis_name="c", subcore_axis_name="s")
    scores = pl.kernel(
        functools.partial(_sc_merge_body, ktop=_KTOP,
                          rows_per_w=rows_per_w),
        mesh=mesh,
        compiler_params=pltpu.CompilerParams(use_tc_tiling_on_sc=False),
        out_type=jax.ShapeDtypeStruct((q_rows,), jnp.float32),
        scratch_types=[
            pltpu.VMEM((ngroups, width, 16), jnp.float32),
            pltpu.VMEM((rows_per_w,), jnp.float32),
        ],
    )(arr)
    return scores / k
